# Initial kernel scaffold; baseline (speedup 1.0000x reference)
import jax
import jax.numpy as jnp
from jax.experimental import pallas as pl


def kernel(x0, x1, hw_x, node_type, edge_path_type, edge_path_len, mask, edge_path_type_r, edge_path_len_r, mask_r, params):
    raise NotImplementedError("write your pallas kernel here")


# speedup vs baseline: 1.4955x; 1.4955x over previous
import jax
import jax.numpy as jnp
from jax.experimental import pallas as pl
from jax.experimental.pallas import tpu as pltpu

_B, _T, _CIN, _D, _H = 2, 24, 32, 128, 4
_NT0, _NT1, _P = 200, 200, 4
_N = _NT0 + _NT1
_DH = _D // _H
_ETYPE, _NTYPE, _LAM, _SC = 3, 2, 0.5, 3
_NP = 512
_TP = 32
_MROW = 2 * _B * _NT0 * _TP
_UROW = 2 * _B * _NT0 * _T
_SROW = _SC * _UROW
_RB = 1920
_SB = 192
_NEG = -1e30


def _cp(*sem):
    return pltpu.CompilerParams(dimension_semantics=sem)


def _conv_kernel(x_ref, w1_ref, w2_ref, w3_ref, b_ref, o_ref):
    x = x_ref[...]
    cin = x.shape[1]
    ws = (w1_ref, w2_ref, w3_ref)
    for i in range(_SC):
        y = b_ref[i]
        for j in range(i + 1):
            if j == 0:
                xs = x
            else:
                xs = jnp.concatenate(
                    [jnp.zeros((j, cin), x.dtype), x[:-j]], axis=0)
            y = y + jnp.dot(xs, ws[i][j], preferred_element_type=jnp.float32)
        o_ref[i] = jnp.tanh(y[:, :_D]) * jax.nn.sigmoid(y[:, _D:])


def _conv_call(xp_flat, w1, w2, w3, bst):
    cin = xp_flat.shape[1]
    nblk = 16
    rb = _MROW // nblk
    return pl.pallas_call(
        _conv_kernel,
        grid=(nblk,),
        in_specs=[
            pl.BlockSpec((rb, cin), lambda i: (i, 0)),
            pl.BlockSpec((1, cin, 2 * _D), lambda i: (0, 0, 0)),
            pl.BlockSpec((2, cin, 2 * _D), lambda i: (0, 0, 0)),
            pl.BlockSpec((3, cin, 2 * _D), lambda i: (0, 0, 0)),
            pl.BlockSpec((_SC, 1, 2 * _D), lambda i: (0, 0, 0)),
        ],
        out_specs=pl.BlockSpec((_SC, rb, _D), lambda i: (0, i, 0)),
        out_shape=jax.ShapeDtypeStruct((_SC, _MROW, _D), jnp.float32),
        compiler_params=_cp("parallel"),
        name="conv_gate",
    )(xp_flat, w1, w2, w3, bst)


def _bd_attn(qkv, scale):
    rows = qkv.shape[0]
    ri = jax.lax.broadcasted_iota(jnp.int32, (_SB, _SB), 0) // _T
    ci = jax.lax.broadcasted_iota(jnp.int32, (_SB, _SB), 1) // _T
    diag = ri == ci
    outs = []
    for s in range(rows // _SB):
        blk = qkv[s * _SB:(s + 1) * _SB]
        q = blk[:, :_D]
        k = blk[:, _D:2 * _D]
        v = blk[:, 2 * _D:]
        sc = jax.lax.dot_general(q, k, (((1,), (1,)), ((), ())),
                                 preferred_element_type=jnp.float32) * scale
        sc = jnp.where(diag, sc, _NEG)
        m = jnp.max(sc, axis=-1, keepdims=True)
        e = jnp.exp(sc - m)
        a = e / jnp.sum(e, axis=-1, keepdims=True)
        outs.append(jnp.dot(a, v, preferred_element_type=jnp.float32))
    return jnp.concatenate(outs, axis=0)


def _lnrow(x, g, b):
    m = jnp.mean(x, axis=-1, keepdims=True)
    c = x - m
    v = jnp.mean(c * c, axis=-1, keepdims=True)
    return c * jax.lax.rsqrt(v + 1e-5) * g + b


def _tlayer_kernel(u_ref, winT, bin_, woutT, bout, ln1g, ln1b,
                   ff1T, bff1, ff2T, bff2, ln2g, ln2b, o_ref):
    x = u_ref[0]
    qkv = jnp.dot(x, winT[0], preferred_element_type=jnp.float32) + bin_[0]
    att = _bd_attn(qkv, 1.0 / jnp.sqrt(jnp.float32(_D)))
    y = jnp.dot(att, woutT[0], preferred_element_type=jnp.float32) + bout[0] + x
    y = _lnrow(y, ln1g[0], ln1b[0])
    h = jnp.maximum(
        jnp.dot(y, ff1T[0], preferred_element_type=jnp.float32) + bff1[0], 0.0)
    z = jnp.dot(h, ff2T[0], preferred_element_type=jnp.float32) + bff2[0] + y
    o_ref[0] = _lnrow(z, ln2g[0], ln2b[0])


def _tlayer_call(u, tp):
    nblk = _UROW // _RB
    wspec = lambda a, b: pl.BlockSpec((1, a, b), lambda s, i: (s, 0, 0))
    return pl.pallas_call(
        _tlayer_kernel,
        grid=(_SC, nblk),
        in_specs=[
            pl.BlockSpec((1, _RB, _D), lambda s, i: (s, i, 0)),
            wspec(_D, 3 * _D), wspec(1, 3 * _D),
            wspec(_D, _D), wspec(1, _D),
            wspec(1, _D), wspec(1, _D),
            wspec(_D, 4 * _D), wspec(1, 4 * _D),
            wspec(4 * _D, _D), wspec(1, _D),
            wspec(1, _D), wspec(1, _D),
        ],
        out_specs=pl.BlockSpec((1, _RB, _D), lambda s, i: (s, i, 0)),
        out_shape=jax.ShapeDtypeStruct((_SC, _UROW, _D), jnp.float32),
        compiler_params=_cp("parallel", "arbitrary"),
        name="tlayer",
    )(u, tp['winT'], tp['bin'], tp['woutT'], tp['bout'], tp['ln1g'],
      tp['ln1b'], tp['ff1T'], tp['bff1'], tp['ff2T'], tp['bff2'],
      tp['ln2g'], tp['ln2b'])


def _mha_kernel(u_ref, winT, bin_, woutT, bout, o_ref):
    x = u_ref[...]
    qkv = jnp.dot(x, winT[...], preferred_element_type=jnp.float32) + bin_[...]
    att = _bd_attn(qkv, 1.0 / jnp.sqrt(jnp.float32(_D)))
    o_ref[...] = (jnp.dot(att, woutT[...], preferred_element_type=jnp.float32)
                  + bout[...])


def _mha_call(s_in, mh):
    nblk = _SROW // _RB
    return pl.pallas_call(
        _mha_kernel,
        grid=(nblk,),
        in_specs=[
            pl.BlockSpec((_RB, _D), lambda i: (i, 0)),
            pl.BlockSpec((_D, 3 * _D), lambda i: (0, 0)),
            pl.BlockSpec((1, 3 * _D), lambda i: (0, 0)),
            pl.BlockSpec((_D, _D), lambda i: (0, 0)),
            pl.BlockSpec((1, _D), lambda i: (0, 0)),
        ],
        out_specs=pl.BlockSpec((_RB, _D), lambda i: (i, 0)),
        out_shape=jax.ShapeDtypeStruct((_SROW, _D), jnp.float32),
        compiler_params=_cp("parallel"),
        name="xscale_mha",
    )(s_in, mh['winT'], mh['bin'], mh['woutT'], mh['bout'])


def _qkv_kernel(x_ref, tm_ref, wq, wk, wv, q_ref, k_ref, v_ref):
    x = x_ref[0]
    tm = tm_ref[...]
    for w, o in ((wq, q_ref), (wk, k_ref), (wv, v_ref)):
        p0 = jnp.dot(x, w[0], preferred_element_type=jnp.float32)
        p1 = jnp.dot(x, w[1], preferred_element_type=jnp.float32)
        o[0] = jnp.where(tm > 0, p1, p0)


def _qkv_call(xf, tm, wq, wk, wv):
    rb = 3072
    nblk = _T * _NP // rb
    osd = jax.ShapeDtypeStruct((_B, _T * _NP, _D), jnp.float32)
    return pl.pallas_call(
        _qkv_kernel,
        grid=(_B, nblk),
        in_specs=[
            pl.BlockSpec((1, rb, _D), lambda b, i: (b, i, 0)),
            pl.BlockSpec((rb, _D), lambda b, i: (i, 0)),
            pl.BlockSpec((_NTYPE, _D, _D), lambda b, i: (0, 0, 0)),
            pl.BlockSpec((_NTYPE, _D, _D), lambda b, i: (0, 0, 0)),
            pl.BlockSpec((_NTYPE, _D, _D), lambda b, i: (0, 0, 0)),
        ],
        out_specs=[pl.BlockSpec((1, rb, _D), lambda b, i: (b, i, 0))] * 3,
        out_shape=[osd, osd, osd],
        compiler_params=_cp("parallel", "arbitrary"),
        name="chgan_qkv",
    )(xf, tm, wq, wk, wv)


def _eprep_kernel(ept_ref, epl_ref, e0_ref, ebw_ref, ebb_ref,
                  mul_ref, add_ref):
    tb = jnp.sum(e0_ref[...] * ebw_ref[...], axis=1, keepdims=True)
    tb1 = tb[1, 0]
    tb2 = tb[2, 0]
    ept = ept_ref[0]
    acc = jnp.zeros(ept.shape[1:], jnp.float32)
    for p_ in range(_P):
        e = ept[p_]
        acc = (acc + jnp.where(e == 1, tb1, jnp.float32(0))
               + jnp.where(e == 2, tb2, jnp.float32(0)))
    bias = acc * (1.0 / _P) + ebb_ref[0, 0]
    dec = jnp.exp(_LAM * (epl_ref[0] - 1.0))
    mul_ref[0] = dec * jax.lax.rsqrt(jnp.float32(_DH))
    add_ref[0] = bias * dec


def _eprep_call(eptp, eplp, e0, ebw, ebb):
    rb = 128
    nblk = _NP // rb
    osd = jax.ShapeDtypeStruct((2, _NP, _NP), jnp.float32)
    return pl.pallas_call(
        _eprep_kernel,
        grid=(2, nblk),
        in_specs=[
            pl.BlockSpec((1, _P, rb, _NP), lambda d, i: (d, 0, i, 0)),
            pl.BlockSpec((1, rb, _NP), lambda d, i: (d, i, 0)),
            pl.BlockSpec((_ETYPE, _D), lambda d, i: (0, 0)),
            pl.BlockSpec((1, _D), lambda d, i: (0, 0)),
            pl.BlockSpec((1, 1), lambda d, i: (0, 0)),
        ],
        out_specs=[pl.BlockSpec((1, rb, _NP), lambda d, i: (d, i, 0))] * 2,
        out_shape=[osd, osd],
        compiler_params=_cp("parallel", "arbitrary"),
        name="edge_prep",
    )(eptp, eplp, e0, ebw, ebb)


def _gattn_kernel(q_ref, k_ref, v_ref, mul_ref, add_ref, msk_ref, o_ref):
    q = q_ref[0, 0]
    k = k_ref[0, 0]
    v = v_ref[0, 0]
    mul = mul_ref[0]
    add = add_ref[0]
    msk = msk_ref[0]
    lane = jax.lax.broadcasted_iota(jnp.int32, (_NP, _D), 1) // _DH
    out = jnp.zeros((_NP, _D), jnp.float32)
    for h in range(_H):
        kh = jnp.where(lane == h, k, 0.0)
        vh = jnp.where(lane == h, v, 0.0)
        sc = jax.lax.dot_general(q, kh, (((1,), (1,)), ((), ())),
                                 preferred_element_type=jnp.float32)
        lg = sc * mul + add
        lg = jnp.where(msk == 0, _NEG, lg)
        m = jnp.max(lg, axis=-1, keepdims=True)
        e = jnp.exp(lg - m)
        a = e / jnp.sum(e, axis=-1, keepdims=True)
        out = out + jnp.dot(a, vh, preferred_element_type=jnp.float32)
    o_ref[0, 0, 0] = out


def _gattn_call(q4, k4, v4, muls, add, mskp):
    qspec = pl.BlockSpec((1, 1, _NP, _D), lambda g, t: (g // 2, t, 0, 0))
    espec = pl.BlockSpec((1, _NP, _NP), lambda g, t: (g % 2, 0, 0))
    return pl.pallas_call(
        _gattn_kernel,
        grid=(2 * _B, _T),
        in_specs=[qspec, qspec, qspec, espec, espec, espec],
        out_specs=pl.BlockSpec((1, 1, 1, _NP, _D),
                               lambda g, t: (g // 2, g % 2, t, 0, 0)),
        out_shape=jax.ShapeDtypeStruct((_B, 2, _T, _NP, _D), jnp.float32),
        compiler_params=_cp("parallel", "arbitrary"),
        name="graph_attn",
    )(q4, k4, v4, muls, add, mskp)


def _chout_kernel(oc_ref, w_ref, b_ref, x_ref, o_ref):
    o_ref[0] = (jnp.dot(oc_ref[0], w_ref[...],
                        preferred_element_type=jnp.float32)
                + b_ref[...] + x_ref[0])


def _chout_call(ocat, woT, ob, xf):
    rb = 3072
    nblk = _T * _NP // rb
    return pl.pallas_call(
        _chout_kernel,
        grid=(_B, nblk),
        in_specs=[
            pl.BlockSpec((1, rb, 2 * _D), lambda b, i: (b, i, 0)),
            pl.BlockSpec((2 * _D, _D), lambda b, i: (0, 0)),
            pl.BlockSpec((1, _D), lambda b, i: (0, 0)),
            pl.BlockSpec((1, rb, _D), lambda b, i: (b, i, 0)),
        ],
        out_specs=pl.BlockSpec((1, rb, _D), lambda b, i: (b, i, 0)),
        out_shape=jax.ShapeDtypeStruct((_B, _T * _NP, _D), jnp.float32),
        compiler_params=_cp("parallel", "arbitrary"),
        name="chgan_out",
    )(ocat, woT, ob, xf)


def _lnr_kernel(z_ref, g_ref, b_ref, o_ref):
    z = z_ref[0, 0]
    m = jnp.mean(z, axis=-1, keepdims=True)
    c = z - m
    v = jnp.mean(c * c, axis=-1, keepdims=True)
    o_ref[0, 0] = jnp.maximum(
        c * jax.lax.rsqrt(v + 1e-5) * g_ref[0] + b_ref[0], 0.0)


def _lnr_call(z, g, b):
    nd = _NT0 * _D
    return pl.pallas_call(
        _lnr_kernel,
        grid=(2, _B),
        in_specs=[
            pl.BlockSpec((1, 1, _T, nd), lambda s, bb: (s, bb, 0, 0)),
            pl.BlockSpec((1, 1, nd), lambda s, bb: (s, 0, 0)),
            pl.BlockSpec((1, 1, nd), lambda s, bb: (s, 0, 0)),
        ],
        out_specs=pl.BlockSpec((1, 1, _T, nd), lambda s, bb: (s, bb, 0, 0)),
        out_shape=jax.ShapeDtypeStruct((2, _B, _T, nd), jnp.float32),
        compiler_params=_cp("parallel", "arbitrary"),
        name="ln_relu",
    )(z, g, b)


def _prep_mswt_params(p):
    convw, convb = [], []
    for i in range(_SC):
        k = i + 1
        w = p['gtu'][i]['w']
        convw.append(jnp.stack([w[:, :, 0, k - 1 - j].T for j in range(k)], 0))
        convb.append(p['gtu'][i]['b'])
    bst = jnp.stack(convb, 0)[:, None, :]
    tl = p['tl']
    st = lambda f: jnp.stack([f(tl[i]) for i in range(_SC)], 0)
    tp = {
        'winT': st(lambda q: q['in_w'].T),
        'bin': st(lambda q: q['in_b'][None, :]),
        'woutT': st(lambda q: q['out_w'].T),
        'bout': st(lambda q: q['out_b'][None, :]),
        'ln1g': st(lambda q: q['ln1_g'][None, :]),
        'ln1b': st(lambda q: q['ln1_b'][None, :]),
        'ff1T': st(lambda q: q['ff1_w'].T),
        'bff1': st(lambda q: q['ff1_b'][None, :]),
        'ff2T': st(lambda q: q['ff2_w'].T),
        'bff2': st(lambda q: q['ff2_b'][None, :]),
        'ln2g': st(lambda q: q['ln2_g'][None, :]),
        'ln2b': st(lambda q: q['ln2_b'][None, :]),
    }
    mh = {'winT': p['mha_in_w'].T, 'bin': p['mha_in_b'][None, :],
          'woutT': p['mha_out_w'].T, 'bout': p['mha_out_b'][None, :]}
    return convw, bst, tp, mh


def _mswt_pass(x_stacked, p):
    cin = x_stacked.shape[-1]
    convw, bst, tp, mh = _prep_mswt_params(p)
    xp = jnp.pad(x_stacked, ((0, 0),) * 3 + ((_TP - _T, 0), (0, 0)))
    xp_flat = xp.reshape(_MROW, cin)
    g = _conv_call(xp_flat, convw[0], convw[1], convw[2], bst)
    g5 = g.reshape(_SC, 2, _B, _NT0, _TP, _D)[:, :, :, :, _TP - _T:, :]
    A = jnp.transpose(g5, (0, 1, 3, 2, 5, 4))
    U = A.reshape(_SC, 2, _T, _B * _NT0, _D)
    U = jnp.transpose(U, (0, 1, 3, 2, 4)).reshape(_SC, _UROW, _D)
    tt = _tlayer_call(U, tp)
    tt6 = tt.reshape(_SC, 2, _B, _NT0, _T, _D)
    s_in = jnp.transpose(tt6, (1, 0, 3, 2, 4, 5)).reshape(_SROW, _D)
    a2 = _mha_call(s_in, mh)
    a6 = a2.reshape(2, _SC, _NT0, _B, _T, _D)
    return jnp.transpose(a6.mean(1), (0, 2, 1, 3, 4))


def _chgan_block(x, p, node_type, ept, epl, msk, ept_r, epl_r, msk_r):
    xpd = jnp.pad(x, ((0, 0), (0, _NP - _N), (0, 0), (0, 0)))
    xf = jnp.transpose(xpd, (0, 2, 1, 3)).reshape(_B, _T * _NP, _D)
    wq, wk, wv = [], [], []
    for dst, w in ((wq, p['Q']), (wk, p['K']), (wv, p['V'])):
        for ty in range(_NTYPE):
            blocks = [[w[ty, h] if h == g else jnp.zeros((_DH, _DH))
                       for g in range(_H)] for h in range(_H)]
            dst.append(jnp.block(blocks))
    wq, wk, wv = jnp.stack(wq), jnp.stack(wk), jnp.stack(wv)
    tmask = jnp.pad((node_type == 1).astype(jnp.float32), (0, _NP - _N))
    tm = jnp.tile(tmask[None, :, None], (_T, 1, _D)).reshape(_T * _NP, _D)
    q, k, v = _qkv_call(xf, tm, wq, wk, wv)

    def padg(ept_, epl_, msk_):
        eptp = jnp.pad(ept_, ((0, _NP - _N), (0, _NP - _N), (0, 0)))
        eplp = jnp.pad(epl_, ((0, _NP - _N), (0, _NP - _N)),
                       constant_values=1.0)
        mskp = jnp.pad(msk_, ((0, _NP - _N), (0, _NP - _N)))
        ar = jnp.arange(_NP)
        mskp = jnp.where((ar[:, None] == ar[None, :]) & (ar[:, None] >= _N),
                         1, mskp)
        return jnp.transpose(eptp, (2, 0, 1)), eplp, mskp
    pa, pb = padg(ept, epl, msk), padg(ept_r, epl_r, msk_r)
    eptp = jnp.stack([pa[0], pb[0]])
    eplp = jnp.stack([pa[1], pb[1]])
    mskp = jnp.stack([pa[2], pb[2]])
    e0 = p['E'].at[0].set(0.0)
    muls, add = _eprep_call(eptp, eplp, e0, p['eb_w'], p['eb_b'][None, :])
    q4 = q.reshape(_B, _T, _NP, _D)
    k4 = k.reshape(_B, _T, _NP, _D)
    v4 = v.reshape(_B, _T, _NP, _D)
    o = _gattn_call(q4, k4, v4, muls, add, mskp)
    ocat = jnp.concatenate([o[:, 0], o[:, 1]], -1).reshape(_B, _T * _NP,
                                                           2 * _D)
    ch = _chout_call(ocat, p['out_w'].T, p['out_b'][None, :], xf)
    ch4 = ch.reshape(_B, _T, _NP, _D)[:, :, :_N]
    return jnp.transpose(ch4, (0, 2, 1, 3))


def kernel(x0, x1, hw_x, node_type, edge_path_type, edge_path_len, mask,
           edge_path_type_r, edge_path_len_r, mask_r, params):
    xs = jnp.stack([x0.reshape(_B, _NT0, _T, _CIN),
                    x1.reshape(_B, _NT1, _T, _CIN)], 0)
    y = _mswt_pass(xs, params['mswt1'])
    x = jnp.concatenate([y[0], y[1]], axis=1)
    ch = _chgan_block(x, params['chgan'], node_type, edge_path_type,
                      edge_path_len, mask, edge_path_type_r, edge_path_len_r,
                      mask_r)
    ch2 = jnp.stack([ch[:, :_NT0], ch[:, _NT0:]], 0)
    y2 = _mswt_pass(ch2, params['mswt2'])
    z = jnp.transpose(y2, (0, 1, 3, 2, 4)).reshape(2, _B, _T, _NT0 * _D)
    g = jnp.stack([params['ln1_g'].reshape(-1), params['ln2_g'].reshape(-1)])
    b = jnp.stack([params['ln1_b'].reshape(-1), params['ln2_b'].reshape(-1)])
    zo = _lnr_call(z, g[:, None, :], b[:, None, :])
    zr = zo.reshape(2, _B, _T, _NT0, _D)
    return jnp.transpose(zr, (1, 0, 3, 2, 4)).reshape(_B, _N, _T, _D)


# speedup vs baseline: 1.7201x; 1.1502x over previous
import jax
import jax.numpy as jnp
from jax.experimental import pallas as pl
from jax.experimental.pallas import tpu as pltpu

_B, _T, _CIN, _D, _H = 2, 24, 32, 128, 4
_NT0, _NT1, _P = 200, 200, 4
_N = _NT0 + _NT1
_DH = _D // _H
_ETYPE, _NTYPE, _LAM, _SC = 3, 2, 0.5, 3
_NP = 512
_UROW = 2 * _B * _NT0 * _T
_SROW = _SC * _UROW
_RB = 1920
_SB = 192
_NEG = -1e30


def _cp(*sem):
    return pltpu.CompilerParams(dimension_semantics=sem)


def _conv_kernel(x_ref, w1_ref, w2_ref, w3_ref, b_ref, o_ref):
    x = x_ref[...]
    rows, cin = x.shape
    trow = jax.lax.broadcasted_iota(jnp.int32, (rows, cin), 0) % _T
    ws = (w1_ref, w2_ref, w3_ref)
    for i in range(_SC):
        y = b_ref[i]
        for j in range(i + 1):
            if j == 0:
                xs = x
            else:
                xs = jnp.concatenate(
                    [jnp.zeros((j, cin), x.dtype), x[:-j]], axis=0)
                xs = jnp.where(trow >= j, xs, 0.0)
            y = y + jnp.dot(xs, ws[i][j], preferred_element_type=jnp.float32)
        o_ref[i] = jnp.tanh(y[:, :_D]) * jax.nn.sigmoid(y[:, _D:])


def _conv_call(x_flat, w1, w2, w3, bst):
    cin = x_flat.shape[1]
    nblk = 10
    rb = _UROW // nblk
    return pl.pallas_call(
        _conv_kernel,
        grid=(nblk,),
        in_specs=[
            pl.BlockSpec((rb, cin), lambda i: (i, 0)),
            pl.BlockSpec((1, cin, 2 * _D), lambda i: (0, 0, 0)),
            pl.BlockSpec((2, cin, 2 * _D), lambda i: (0, 0, 0)),
            pl.BlockSpec((3, cin, 2 * _D), lambda i: (0, 0, 0)),
            pl.BlockSpec((_SC, 1, 2 * _D), lambda i: (0, 0, 0)),
        ],
        out_specs=pl.BlockSpec((_SC, rb, _D), lambda i: (0, i, 0)),
        out_shape=jax.ShapeDtypeStruct((_SC, _UROW, _D), jnp.float32),
        compiler_params=_cp("parallel"),
        name="conv_gate",
    )(x_flat, w1, w2, w3, bst)


def _bd_attn(qkv, scale):
    rows = qkv.shape[0]
    ri = jax.lax.broadcasted_iota(jnp.int32, (_SB, _SB), 0) // _T
    ci = jax.lax.broadcasted_iota(jnp.int32, (_SB, _SB), 1) // _T
    diag = ri == ci
    outs = []
    for s in range(rows // _SB):
        blk = qkv[s * _SB:(s + 1) * _SB]
        q = blk[:, :_D]
        k = blk[:, _D:2 * _D]
        v = blk[:, 2 * _D:]
        sc = jax.lax.dot_general(q, k, (((1,), (1,)), ((), ())),
                                 preferred_element_type=jnp.float32) * scale
        sc = jnp.where(diag, sc, _NEG)
        m = jnp.max(sc, axis=-1, keepdims=True)
        e = jnp.exp(sc - m)
        a = e / jnp.sum(e, axis=-1, keepdims=True)
        outs.append(jnp.dot(a, v, preferred_element_type=jnp.float32))
    return jnp.concatenate(outs, axis=0)


def _lnrow(x, g, b):
    m = jnp.mean(x, axis=-1, keepdims=True)
    c = x - m
    v = jnp.mean(c * c, axis=-1, keepdims=True)
    return c * jax.lax.rsqrt(v + 1e-5) * g + b


def _tlayer_kernel(u_ref, winT, bin_, woutT, bout, ln1g, ln1b,
                   ff1T, bff1, ff2T, bff2, ln2g, ln2b, o_ref):
    x = u_ref[0]
    qkv = jnp.dot(x, winT[0], preferred_element_type=jnp.float32) + bin_[0]
    att = _bd_attn(qkv, 1.0 / jnp.sqrt(jnp.float32(_D)))
    y = jnp.dot(att, woutT[0], preferred_element_type=jnp.float32) + bout[0] + x
    y = _lnrow(y, ln1g[0], ln1b[0])
    h = jnp.maximum(
        jnp.dot(y, ff1T[0], preferred_element_type=jnp.float32) + bff1[0], 0.0)
    z = jnp.dot(h, ff2T[0], preferred_element_type=jnp.float32) + bff2[0] + y
    o_ref[0] = _lnrow(z, ln2g[0], ln2b[0])


def _tlayer_call(u, tp):
    nblk = _UROW // _RB
    wspec = lambda a, b: pl.BlockSpec((1, a, b), lambda s, i: (s, 0, 0))
    return pl.pallas_call(
        _tlayer_kernel,
        grid=(_SC, nblk),
        in_specs=[
            pl.BlockSpec((1, _RB, _D), lambda s, i: (s, i, 0)),
            wspec(_D, 3 * _D), wspec(1, 3 * _D),
            wspec(_D, _D), wspec(1, _D),
            wspec(1, _D), wspec(1, _D),
            wspec(_D, 4 * _D), wspec(1, 4 * _D),
            wspec(4 * _D, _D), wspec(1, _D),
            wspec(1, _D), wspec(1, _D),
        ],
        out_specs=pl.BlockSpec((1, _RB, _D), lambda s, i: (s, i, 0)),
        out_shape=jax.ShapeDtypeStruct((_SC, _UROW, _D), jnp.float32),
        compiler_params=_cp("parallel", "arbitrary"),
        name="tlayer",
    )(u, tp['winT'], tp['bin'], tp['woutT'], tp['bout'], tp['ln1g'],
      tp['ln1b'], tp['ff1T'], tp['bff1'], tp['ff2T'], tp['bff2'],
      tp['ln2g'], tp['ln2b'])


def _mha_kernel(u_ref, winT, bin_, woutT, bout, o_ref):
    x = u_ref[0]
    qkv = jnp.dot(x, winT[...], preferred_element_type=jnp.float32) + bin_[...]
    att = _bd_attn(qkv, 1.0 / jnp.sqrt(jnp.float32(_D)))
    res = (jnp.dot(att, woutT[...], preferred_element_type=jnp.float32)
           + bout[...]) * (1.0 / _SC)
    i = pl.program_id(2)

    @pl.when(i == 0)
    def _():
        o_ref[...] = res

    @pl.when(i > 0)
    def _():
        o_ref[...] = o_ref[...] + res


def _mha_call(tt, mh):
    nblk = _UROW // _RB
    hb = nblk // 2
    wsp = lambda a, b: pl.BlockSpec((a, b), lambda c, i, s: (0, 0))
    return pl.pallas_call(
        _mha_kernel,
        grid=(2, hb, _SC),
        in_specs=[
            pl.BlockSpec((1, _RB, _D), lambda c, i, s: (s, c * hb + i, 0)),
            wsp(_D, 3 * _D), wsp(1, 3 * _D), wsp(_D, _D), wsp(1, _D),
        ],
        out_specs=pl.BlockSpec((_RB, _D), lambda c, i, s: (c * hb + i, 0)),
        out_shape=jax.ShapeDtypeStruct((_UROW, _D), jnp.float32),
        compiler_params=_cp("parallel", "arbitrary", "arbitrary"),
        name="xscale_mha",
    )(tt, mh['winT'], mh['bin'], mh['woutT'], mh['bout'])


def _qkv_kernel(x_ref, tm_ref, wq, wk, wv, q_ref, k_ref, v_ref):
    x = x_ref[0]
    tm = tm_ref[...]
    for w, o in ((wq, q_ref), (wk, k_ref), (wv, v_ref)):
        p0 = jnp.dot(x, w[0], preferred_element_type=jnp.float32)
        p1 = jnp.dot(x, w[1], preferred_element_type=jnp.float32)
        o[0] = jnp.where(tm > 0, p1, p0)


def _qkv_call(xf, tm, wq, wk, wv):
    rb = 3072
    nblk = _T * _NP // rb
    osd = jax.ShapeDtypeStruct((_B, _T * _NP, _D), jnp.float32)
    return pl.pallas_call(
        _qkv_kernel,
        grid=(_B, nblk),
        in_specs=[
            pl.BlockSpec((1, rb, _D), lambda b, i: (b, i, 0)),
            pl.BlockSpec((rb, _D), lambda b, i: (i, 0)),
            pl.BlockSpec((_NTYPE, _D, _D), lambda b, i: (0, 0, 0)),
            pl.BlockSpec((_NTYPE, _D, _D), lambda b, i: (0, 0, 0)),
            pl.BlockSpec((_NTYPE, _D, _D), lambda b, i: (0, 0, 0)),
        ],
        out_specs=[pl.BlockSpec((1, rb, _D), lambda b, i: (b, i, 0))] * 3,
        out_shape=[osd, osd, osd],
        compiler_params=_cp("parallel", "arbitrary"),
        name="chgan_qkv",
    )(xf, tm, wq, wk, wv)


def _eprep_kernel(ept_ref, epl_ref, msk_ref, e0_ref, ebw_ref, ebb_ref,
                  mul_ref, add_ref):
    tb = jnp.sum(e0_ref[...] * ebw_ref[...], axis=1, keepdims=True)
    tb1 = tb[1, 0]
    tb2 = tb[2, 0]
    ept = ept_ref[0]
    acc = jnp.zeros(ept.shape[1:], jnp.float32)
    for p_ in range(_P):
        e = ept[p_]
        acc = (acc + jnp.where(e == 1, tb1, jnp.float32(0))
               + jnp.where(e == 2, tb2, jnp.float32(0)))
    bias = acc * (1.0 / _P) + ebb_ref[0, 0]
    dec = jnp.exp(_LAM * (epl_ref[0] - 1.0))
    mul_ref[0] = dec * jax.lax.rsqrt(jnp.float32(_DH))
    add_ref[0] = jnp.where(msk_ref[0] == 0, _NEG, bias * dec)


def _eprep_call(eptp, eplp, mskp, e0, ebw, ebb):
    rb = 128
    nblk = _NP // rb
    osd = jax.ShapeDtypeStruct((2, _NP, _NP), jnp.float32)
    return pl.pallas_call(
        _eprep_kernel,
        grid=(2, nblk),
        in_specs=[
            pl.BlockSpec((1, _P, rb, _NP), lambda d, i: (d, 0, i, 0)),
            pl.BlockSpec((1, rb, _NP), lambda d, i: (d, i, 0)),
            pl.BlockSpec((1, rb, _NP), lambda d, i: (d, i, 0)),
            pl.BlockSpec((_ETYPE, _D), lambda d, i: (0, 0)),
            pl.BlockSpec((1, _D), lambda d, i: (0, 0)),
            pl.BlockSpec((1, 1), lambda d, i: (0, 0)),
        ],
        out_specs=[pl.BlockSpec((1, rb, _NP), lambda d, i: (d, i, 0))] * 2,
        out_shape=[osd, osd],
        compiler_params=_cp("parallel", "arbitrary"),
        name="edge_prep",
    )(eptp, eplp, mskp, e0, ebw, ebb)


def _gattn_kernel(q_ref, k_ref, v_ref, mul_ref, add_ref, o_ref):
    q = q_ref[0, 0]
    k = k_ref[0, 0]
    v = v_ref[0, 0]
    mul = mul_ref[0]
    add = add_ref[0]
    lane = jax.lax.broadcasted_iota(jnp.int32, (_NP, _D), 1) // _DH
    out = jnp.zeros((_NP, _D), jnp.float32)
    for h in range(_H):
        kh = jnp.where(lane == h, k, 0.0)
        vh = jnp.where(lane == h, v, 0.0)
        sc = jax.lax.dot_general(q, kh, (((1,), (1,)), ((), ())),
                                 preferred_element_type=jnp.float32)
        lg = sc * mul + add
        m = jnp.max(lg, axis=-1, keepdims=True)
        e = jnp.exp(lg - m)
        a = e / jnp.sum(e, axis=-1, keepdims=True)
        out = out + jnp.dot(a, vh, preferred_element_type=jnp.float32)
    o_ref[0, 0, 0] = out


def _gattn_call(q4, k4, v4, muls, add):
    gi = lambda c, i: (c * 2 * _T + i) // _T
    ti = lambda c, i: (c * 2 * _T + i) % _T
    qspec = pl.BlockSpec((1, 1, _NP, _D),
                         lambda c, i: (gi(c, i) // 2, ti(c, i), 0, 0))
    espec = pl.BlockSpec((1, _NP, _NP), lambda c, i: (gi(c, i) % 2, 0, 0))
    return pl.pallas_call(
        _gattn_kernel,
        grid=(2, 2 * _T),
        in_specs=[qspec, qspec, qspec, espec, espec],
        out_specs=pl.BlockSpec(
            (1, 1, 1, _NP, _D),
            lambda c, i: (gi(c, i) // 2, gi(c, i) % 2, ti(c, i), 0, 0)),
        out_shape=jax.ShapeDtypeStruct((_B, 2, _T, _NP, _D), jnp.float32),
        compiler_params=_cp("parallel", "arbitrary"),
        name="graph_attn",
    )(q4, k4, v4, muls, add)


def _chout_kernel(of_ref, or_ref, w_ref, b_ref, x_ref, o_ref):
    tb = of_ref.shape[2]
    o0 = of_ref[0, 0].reshape(tb * _NP, _D)
    o1 = or_ref[0, 0].reshape(tb * _NP, _D)
    o_ref[0] = (jnp.dot(o0, w_ref[0], preferred_element_type=jnp.float32)
                + jnp.dot(o1, w_ref[1], preferred_element_type=jnp.float32)
                + b_ref[...] + x_ref[0])


def _chout_call(o5, w2, ob, xf):
    tb = 6
    nblk = _T // tb
    dspec = lambda d: pl.BlockSpec((1, 1, tb, _NP, _D),
                                   lambda b, i: (b, d, i, 0, 0))
    return pl.pallas_call(
        _chout_kernel,
        grid=(_B, nblk),
        in_specs=[
            dspec(0), dspec(1),
            pl.BlockSpec((2, _D, _D), lambda b, i: (0, 0, 0)),
            pl.BlockSpec((1, _D), lambda b, i: (0, 0)),
            pl.BlockSpec((1, tb * _NP, _D), lambda b, i: (b, i, 0)),
        ],
        out_specs=pl.BlockSpec((1, tb * _NP, _D), lambda b, i: (b, i, 0)),
        out_shape=jax.ShapeDtypeStruct((_B, _T * _NP, _D), jnp.float32),
        compiler_params=_cp("parallel", "arbitrary"),
        name="chgan_out",
    )(o5, o5, w2, ob, xf)


def _lnr_kernel(z_ref, g_ref, b_ref, o_ref):
    z = z_ref[0, 0]
    m = jnp.mean(z, axis=-1, keepdims=True)
    c = z - m
    v = jnp.mean(c * c, axis=-1, keepdims=True)
    o_ref[0, 0] = jnp.maximum(
        c * jax.lax.rsqrt(v + 1e-5) * g_ref[0] + b_ref[0], 0.0)


def _lnr_call(z, g, b):
    nd = _NT0 * _D
    return pl.pallas_call(
        _lnr_kernel,
        grid=(2, _B),
        in_specs=[
            pl.BlockSpec((1, 1, _T, nd), lambda s, bb: (s, bb, 0, 0)),
            pl.BlockSpec((1, 1, nd), lambda s, bb: (s, 0, 0)),
            pl.BlockSpec((1, 1, nd), lambda s, bb: (s, 0, 0)),
        ],
        out_specs=pl.BlockSpec((1, 1, _T, nd), lambda s, bb: (s, bb, 0, 0)),
        out_shape=jax.ShapeDtypeStruct((2, _B, _T, nd), jnp.float32),
        compiler_params=_cp("parallel", "arbitrary"),
        name="ln_relu",
    )(z, g, b)


def _prep_mswt_params(p):
    convw, convb = [], []
    for i in range(_SC):
        k = i + 1
        w = p['gtu'][i]['w']
        convw.append(jnp.stack([w[:, :, 0, k - 1 - j].T for j in range(k)], 0))
        convb.append(p['gtu'][i]['b'])
    bst = jnp.stack(convb, 0)[:, None, :]
    tl = p['tl']
    st = lambda f: jnp.stack([f(tl[i]) for i in range(_SC)], 0)
    tp = {
        'winT': st(lambda q: q['in_w'].T),
        'bin': st(lambda q: q['in_b'][None, :]),
        'woutT': st(lambda q: q['out_w'].T),
        'bout': st(lambda q: q['out_b'][None, :]),
        'ln1g': st(lambda q: q['ln1_g'][None, :]),
        'ln1b': st(lambda q: q['ln1_b'][None, :]),
        'ff1T': st(lambda q: q['ff1_w'].T),
        'bff1': st(lambda q: q['ff1_b'][None, :]),
        'ff2T': st(lambda q: q['ff2_w'].T),
        'bff2': st(lambda q: q['ff2_b'][None, :]),
        'ln2g': st(lambda q: q['ln2_g'][None, :]),
        'ln2b': st(lambda q: q['ln2_b'][None, :]),
    }
    mh = {'winT': p['mha_in_w'].T, 'bin': p['mha_in_b'][None, :],
          'woutT': p['mha_out_w'].T, 'bout': p['mha_out_b'][None, :]}
    return convw, bst, tp, mh


def _mswt_pass(x_flat, p):
    cin = x_flat.shape[-1]
    convw, bst, tp, mh = _prep_mswt_params(p)
    g = _conv_call(x_flat, convw[0], convw[1], convw[2], bst)
    g6 = g.reshape(_SC, 2, _B, _NT0, _T, _D)
    A = jnp.transpose(g6, (0, 1, 3, 2, 5, 4))
    ro = A.reshape(_SC, 2, _T, _B, _NT0, _D)
    U = jnp.transpose(ro, (0, 1, 4, 3, 2, 5)).reshape(_SC, _UROW, _D)
    tt = _tlayer_call(U, tp)
    return _mha_call(tt, mh)


def _chgan_block(xf, p, node_type, ept, epl, msk, ept_r, epl_r, msk_r):
    wq, wk, wv = [], [], []
    for dst, w in ((wq, p['Q']), (wk, p['K']), (wv, p['V'])):
        for ty in range(_NTYPE):
            blocks = [[w[ty, h] if h == g else jnp.zeros((_DH, _DH))
                       for g in range(_H)] for h in range(_H)]
            dst.append(jnp.block(blocks))
    wq, wk, wv = jnp.stack(wq), jnp.stack(wk), jnp.stack(wv)
    tmask = jnp.pad((node_type == 1).astype(jnp.float32), (0, _NP - _N))
    tm = jnp.tile(tmask[None, :, None], (_T, 1, _D)).reshape(_T * _NP, _D)
    q, k, v = _qkv_call(xf, tm, wq, wk, wv)

    def padg(ept_, epl_, msk_):
        eptp = jnp.pad(ept_, ((0, _NP - _N), (0, _NP - _N), (0, 0)))
        eplp = jnp.pad(epl_, ((0, _NP - _N), (0, _NP - _N)),
                       constant_values=1.0)
        mskp = jnp.pad(msk_, ((0, _NP - _N), (0, _NP - _N)))
        ar = jnp.arange(_NP)
        mskp = jnp.where((ar[:, None] == ar[None, :]) & (ar[:, None] >= _N),
                         1, mskp)
        return jnp.transpose(eptp, (2, 0, 1)), eplp, mskp
    pa, pb = padg(ept, epl, msk), padg(ept_r, epl_r, msk_r)
    eptp = jnp.stack([pa[0], pb[0]])
    eplp = jnp.stack([pa[1], pb[1]])
    mskp = jnp.stack([pa[2], pb[2]])
    e0 = p['E'].at[0].set(0.0)
    muls, add = _eprep_call(eptp, eplp, mskp, e0, p['eb_w'],
                            p['eb_b'][None, :])
    q4 = q.reshape(_B, _T, _NP, _D)
    k4 = k.reshape(_B, _T, _NP, _D)
    v4 = v.reshape(_B, _T, _NP, _D)
    o = _gattn_call(q4, k4, v4, muls, add)
    woT = p['out_w'].T
    w2 = jnp.stack([woT[:_D], woT[_D:]], 0)
    return _chout_call(o, w2, p['out_b'][None, :], xf)


def kernel(x0, x1, hw_x, node_type, edge_path_type, edge_path_len, mask,
           edge_path_type_r, edge_path_len_r, mask_r, params):
    xs = jnp.stack([x0.reshape(_B, _NT0, _T, _CIN),
                    x1.reshape(_B, _NT1, _T, _CIN)], 0)
    y = _mswt_pass(xs.reshape(_UROW, _CIN), params['mswt1'])
    y5 = y.reshape(2, _NT0, _B, _T, _D)
    xg = jnp.transpose(y5, (2, 3, 0, 1, 4)).reshape(_B, _T, _N, _D)
    xf = jnp.pad(xg, ((0, 0), (0, 0), (0, _NP - _N), (0, 0)))
    xf = xf.reshape(_B, _T * _NP, _D)
    ch = _chgan_block(xf, params['chgan'], node_type, edge_path_type,
                      edge_path_len, mask, edge_path_type_r, edge_path_len_r,
                      mask_r)
    ch5 = ch.reshape(_B, _T, _NP, _D)[:, :, :_N].reshape(_B, _T, 2, _NT0, _D)
    x2 = jnp.transpose(ch5, (2, 0, 3, 1, 4)).reshape(_UROW, _D)
    y2 = _mswt_pass(x2, params['mswt2'])
    z5 = y2.reshape(2, _NT0, _B, _T, _D)
    z = jnp.transpose(z5, (0, 2, 3, 1, 4)).reshape(2, _B, _T, _NT0 * _D)
    g = jnp.stack([params['ln1_g'].reshape(-1), params['ln2_g'].reshape(-1)])
    b = jnp.stack([params['ln1_b'].reshape(-1), params['ln2_b'].reshape(-1)])
    zo = _lnr_call(z, g[:, None, :], b[:, None, :])
    zr = zo.reshape(2, _B, _T, _NT0, _D)
    return jnp.transpose(zr, (1, 0, 3, 2, 4)).reshape(_B, _N, _T, _D)
